# Initial kernel scaffold; baseline (speedup 1.0000x reference)
import jax
import jax.numpy as jnp
from jax.experimental import pallas as pl


def kernel(x, timestamp, te):
    raise NotImplementedError("write your pallas kernel here")


# speedup vs baseline: 2.1175x; 2.1175x over previous
import functools

import jax
import jax.numpy as jnp
from jax import lax
from jax.experimental import pallas as pl
from jax.experimental.pallas import tpu as pltpu
from jax.experimental.pallas import tpu_sc as plsc

D = 64
NC = 2
NS = 16
NW = NC * NS
CHUNK = 512
GSUB = 128
NGATH = CHUNK // GSUB


def _sc_body(nchunks, x_hbm, idx_hbm, te_hbm, out_hbm,
             idx_v, emb_v, x_v, sem):
    cid = lax.axis_index("c")
    sid = lax.axis_index("s")
    wid = sid * NC + cid
    chunk0 = wid * nchunks

    def run_chunk(ci, _):
        rbase = (chunk0 + ci) * CHUNK
        pltpu.sync_copy(idx_hbm.at[pl.ds((chunk0 + ci) * NGATH, NGATH)],
                        idx_v)
        copies = [
            pltpu.async_copy(te_hbm.at[idx_v.at[j]],
                             emb_v.at[pl.ds(j * GSUB, GSUB)], sem)
            for j in range(NGATH)
        ]
        copies.append(pltpu.async_copy(x_hbm.at[pl.ds(rbase, CHUNK)],
                                       x_v, sem))
        for c in copies:
            c.wait()

        def add_row(i, _):
            for k in range(D // 16):
                sl = pl.ds(k * 16, 16)
                x_v[i, sl] = x_v[i, sl] + emb_v[i, sl]
            return 0

        lax.fori_loop(0, CHUNK, add_row, 0)
        pltpu.sync_copy(x_v, out_hbm.at[pl.ds(rbase, CHUNK)])
        return 0

    lax.fori_loop(0, nchunks, run_chunk, 0)


@functools.partial(jax.jit, static_argnames=("n",))
def _run(x2d, idx2d, te, n):
    nchunks = n // (NW * CHUNK)
    body = functools.partial(_sc_body, nchunks)
    return pl.kernel(
        body,
        out_type=jax.ShapeDtypeStruct((n, D), jnp.float32),
        mesh=plsc.VectorSubcoreMesh(core_axis_name="c", subcore_axis_name="s"),
        scratch_types=[
            pltpu.VMEM((NGATH, GSUB), jnp.int32),
            pltpu.VMEM((CHUNK, D), jnp.float32),
            pltpu.VMEM((CHUNK, D), jnp.float32),
            pltpu.SemaphoreType.DMA,
        ],
        compiler_params=pltpu.CompilerParams(use_tc_tiling_on_sc=False),
    )(x2d, idx2d, te)


def kernel(x, timestamp, te):
    b, h, d = x.shape
    n = b * h
    x2d = x.reshape(n, d)
    idx2d = timestamp.astype(jnp.int32).reshape(n // GSUB, GSUB)
    out = _run(x2d, idx2d, te, n)
    return out.reshape(b, h, d)


# speedup vs baseline: 2.1356x; 1.0086x over previous
import functools

import jax
import jax.numpy as jnp
from jax import lax
from jax.experimental import pallas as pl
from jax.experimental.pallas import tpu as pltpu
from jax.experimental.pallas import tpu_sc as plsc

D = 64
NC = 2
NS = 16
NW = NC * NS
CHUNK = 256
GSUB = 128
NGATH = CHUNK // GSUB


def _sc_body(nchunks, x_hbm, idx_hbm, te_hbm, out_hbm,
             idx_all, emb_a, emb_b, x_a, x_b, out_a, out_b,
             dma_a, dma_b, st_a, st_b):
    cid = lax.axis_index("c")
    sid = lax.axis_index("s")
    wid = sid * NC + cid
    chunk0 = wid * nchunks

    def issue_load(ci, emb_v, x_v, sem):
        for j in range(NGATH):
            pltpu.async_copy(te_hbm.at[idx_all.at[ci * NGATH + j]],
                             emb_v.at[pl.ds(j * GSUB, GSUB)], sem)
        pltpu.async_copy(x_hbm.at[pl.ds((chunk0 + ci) * CHUNK, CHUNK)],
                         x_v, sem)

    def wait_load(emb_v, x_v, sem):
        pltpu.make_async_copy(te_hbm.at[pl.ds(0, CHUNK)], emb_v, sem).wait()
        pltpu.make_async_copy(x_hbm.at[pl.ds(0, CHUNK)], x_v, sem).wait()

    def issue_store(ci, out_v, sem):
        pltpu.async_copy(out_v,
                         out_hbm.at[pl.ds((chunk0 + ci) * CHUNK, CHUNK)], sem)

    def wait_store(out_v, sem):
        pltpu.make_async_copy(out_v, out_hbm.at[pl.ds(0, CHUNK)], sem).wait()

    def compute(emb_v, x_v, out_v):
        @plsc.parallel_loop(0, CHUNK, unroll=4)
        def _(i):
            for k in range(D // 16):
                sl = pl.ds(k * 16, 16)
                out_v[i, sl] = x_v[i, sl] + emb_v[i, sl]

    pltpu.sync_copy(idx_hbm.at[pl.ds(chunk0 * NGATH, nchunks * NGATH)],
                    idx_all)
    issue_load(0, emb_a, x_a, dma_a)
    issue_load(1, emb_b, x_b, dma_b)

    def run_pair(p, _):
        c0 = 2 * p
        wait_load(emb_a, x_a, dma_a)

        @pl.when(p > 0)
        def _():
            wait_store(out_a, st_a)

        compute(emb_a, x_a, out_a)
        issue_store(c0, out_a, st_a)

        @pl.when(c0 + 2 < nchunks)
        def _():
            issue_load(c0 + 2, emb_a, x_a, dma_a)

        wait_load(emb_b, x_b, dma_b)

        @pl.when(p > 0)
        def _():
            wait_store(out_b, st_b)

        compute(emb_b, x_b, out_b)
        issue_store(c0 + 1, out_b, st_b)

        @pl.when(c0 + 3 < nchunks)
        def _():
            issue_load(c0 + 3, emb_b, x_b, dma_b)

        return 0

    lax.fori_loop(0, nchunks // 2, run_pair, 0)
    wait_store(out_a, st_a)
    wait_store(out_b, st_b)


@functools.partial(jax.jit, static_argnames=("n",))
def _run(x2d, idx2d, te, n):
    nchunks = n // (NW * CHUNK)
    body = functools.partial(_sc_body, nchunks)
    return pl.kernel(
        body,
        out_type=jax.ShapeDtypeStruct((n, D), jnp.float32),
        mesh=plsc.VectorSubcoreMesh(core_axis_name="c", subcore_axis_name="s"),
        scratch_types=[
            pltpu.VMEM((nchunks * NGATH, GSUB), jnp.int32),
            pltpu.VMEM((CHUNK, D), jnp.float32),
            pltpu.VMEM((CHUNK, D), jnp.float32),
            pltpu.VMEM((CHUNK, D), jnp.float32),
            pltpu.VMEM((CHUNK, D), jnp.float32),
            pltpu.VMEM((CHUNK, D), jnp.float32),
            pltpu.VMEM((CHUNK, D), jnp.float32),
            pltpu.SemaphoreType.DMA,
            pltpu.SemaphoreType.DMA,
            pltpu.SemaphoreType.DMA,
            pltpu.SemaphoreType.DMA,
        ],
        compiler_params=pltpu.CompilerParams(use_tc_tiling_on_sc=False),
    )(x2d, idx2d, te)


def kernel(x, timestamp, te):
    b, h, d = x.shape
    n = b * h
    x2d = x.reshape(n, d)
    idx2d = timestamp.astype(jnp.int32).reshape(n // GSUB, GSUB)
    out = _run(x2d, idx2d, te, n)
    return out.reshape(b, h, d)


# speedup vs baseline: 2.5954x; 1.2153x over previous
import functools

import jax
import jax.numpy as jnp
from jax import lax
from jax.experimental import pallas as pl
from jax.experimental.pallas import tpu as pltpu
from jax.experimental.pallas import tpu_sc as plsc

D = 64
V = 446
NC = 2
NS = 16
NW = NC * NS
CHUNK = 256


def _sc_body(nchunks, x_hbm, idx_hbm, te_hbm, out_hbm,
             te_v, idx_all, x_a, x_b, out_a, out_b,
             dma_a, dma_b, st_a, st_b):
    cid = lax.axis_index("c")
    sid = lax.axis_index("s")
    wid = sid * NC + cid
    row0 = wid * nchunks * CHUNK

    def issue_load(ci, x_v, sem):
        pltpu.async_copy(x_hbm.at[pl.ds(row0 + ci * CHUNK, CHUNK)], x_v, sem)

    def wait_load(x_v, sem):
        pltpu.make_async_copy(x_hbm.at[pl.ds(0, CHUNK)], x_v, sem).wait()

    def issue_store(ci, out_v, sem):
        pltpu.async_copy(out_v,
                         out_hbm.at[pl.ds(row0 + ci * CHUNK, CHUNK)], sem)

    def wait_store(out_v, sem):
        pltpu.make_async_copy(out_v, out_hbm.at[pl.ds(0, CHUNK)], sem).wait()

    def compute(ci, x_v, out_v):
        @plsc.parallel_loop(0, CHUNK // 16, unroll=1)
        def _(g):
            tvec = idx_all[pl.ds(ci * CHUNK + g * 16, 16)]
            for r in range(16):
                t = tvec[r]
                i = g * 16 + r
                for k in range(D // 16):
                    sl = pl.ds(k * 16, 16)
                    out_v[i, sl] = x_v[i, sl] + te_v[t, sl]

    pltpu.sync_copy(te_hbm, te_v)
    pltpu.sync_copy(idx_hbm.at[pl.ds(row0, nchunks * CHUNK)], idx_all)
    issue_load(0, x_a, dma_a)
    issue_load(1, x_b, dma_b)

    def run_pair(p, _):
        c0 = 2 * p
        wait_load(x_a, dma_a)

        @pl.when(p > 0)
        def _():
            wait_store(out_a, st_a)

        compute(c0, x_a, out_a)
        issue_store(c0, out_a, st_a)

        @pl.when(c0 + 2 < nchunks)
        def _():
            issue_load(c0 + 2, x_a, dma_a)

        wait_load(x_b, dma_b)

        @pl.when(p > 0)
        def _():
            wait_store(out_b, st_b)

        compute(c0 + 1, x_b, out_b)
        issue_store(c0 + 1, out_b, st_b)

        @pl.when(c0 + 3 < nchunks)
        def _():
            issue_load(c0 + 3, x_b, dma_b)

        return 0

    lax.fori_loop(0, nchunks // 2, run_pair, 0)
    wait_store(out_a, st_a)
    wait_store(out_b, st_b)


@functools.partial(jax.jit, static_argnames=("n",))
def _run(x2d, idx, te, n):
    nchunks = n // (NW * CHUNK)
    body = functools.partial(_sc_body, nchunks)
    return pl.kernel(
        body,
        out_type=jax.ShapeDtypeStruct((n, D), jnp.float32),
        mesh=plsc.VectorSubcoreMesh(core_axis_name="c", subcore_axis_name="s"),
        scratch_types=[
            pltpu.VMEM((V, D), jnp.float32),
            pltpu.VMEM((nchunks * CHUNK,), jnp.int32),
            pltpu.VMEM((CHUNK, D), jnp.float32),
            pltpu.VMEM((CHUNK, D), jnp.float32),
            pltpu.VMEM((CHUNK, D), jnp.float32),
            pltpu.VMEM((CHUNK, D), jnp.float32),
            pltpu.SemaphoreType.DMA,
            pltpu.SemaphoreType.DMA,
            pltpu.SemaphoreType.DMA,
            pltpu.SemaphoreType.DMA,
        ],
        compiler_params=pltpu.CompilerParams(use_tc_tiling_on_sc=False),
    )(x2d, idx, te)


def kernel(x, timestamp, te):
    b, h, d = x.shape
    n = b * h
    x2d = x.reshape(n, d)
    idx = timestamp.astype(jnp.int32).reshape(n)
    out = _run(x2d, idx, te, n)
    return out.reshape(b, h, d)
